# baseline (speedup 1.0000x reference)
import functools

import jax
import jax.numpy as jnp
from jax import lax
from jax.experimental import pallas as pl
from jax.experimental.pallas import tpu as pltpu
from jax.experimental.pallas import tpu_sc as plsc

_NC = 2
_NS = 16
_NW = _NC * _NS
_BW = 128
_NBUF = 2
_NGB = 4


@functools.lru_cache(maxsize=None)
def _make_kernel(batch, hist, d_embed):
    nwin = batch // _BW
    wpt = nwin // _NW
    nblk = wpt * hist
    td_n = d_embed // 8
    mesh = plsc.VectorSubcoreMesh(core_axis_name="c", subcore_axis_name="s")

    @functools.partial(
        pl.kernel,
        mesh=mesh,
        out_type=jax.ShapeDtypeStruct((hist, td_n, nwin, 8, _BW), jnp.float32),
        scratch_types=[
            pltpu.VMEM((wpt, hist, _BW), jnp.int32),
            *[pltpu.VMEM((_BW, d_embed), jnp.float32) for _ in range(_NGB)],
            *[pltpu.VMEM((td_n, 8, _BW), jnp.float32) for _ in range(_NBUF)],
            *[pltpu.SemaphoreType.DMA for _ in range(_NGB + _NBUF)],
        ],
        compiler_params=pltpu.CompilerParams(
            use_tc_tiling_on_sc=False, needs_layout_passes=False),
    )
    def gather_kernel(idx_hbm, table_hbm, out_hbm, idx_v, *bufs_and_sems):
        gbufs = bufs_and_sems[:_NGB]
        tbufs = bufs_and_sems[_NGB:_NGB + _NBUF]
        gsems = bufs_and_sems[_NGB + _NBUF:2 * _NGB + _NBUF]
        ssems = bufs_and_sems[2 * _NGB + _NBUF:]
        wid = lax.axis_index("s") * _NC + lax.axis_index("c")
        pltpu.sync_copy(idx_hbm.at[pl.ds(wid * wpt, wpt)], idx_v)
        lanes = lax.iota(jnp.int32, 16)

        def fire(g, j):
            bw = j // hist
            h = j - bw * hist
            pltpu.async_copy(table_hbm.at[idx_v.at[bw, h]], gbufs[g], gsems[g])

        def transpose(g, b):
            gbuf, tbuf = gbufs[g], tbufs[b]

            def kb_body(kb, carry):
                rows = lanes + kb * 16
                for d0 in range(0, d_embed, 16):
                    vs = [
                        plsc.load_gather(
                            gbuf, [rows, jnp.full((16,), d0 + dd, jnp.int32)])
                        for dd in range(16)
                    ]
                    for dd, v in enumerate(vs):
                        d = d0 + dd
                        tbuf[d // 8, d % 8, pl.ds(kb * 16, 16)] = v
                return carry

            lax.fori_loop(0, _BW // 16, kb_body, 0)

        def step(j, g, b, fire_ahead, wait_prev):
            bw = j // hist
            h = j - bw * hist
            bwg = wid * wpt + bw
            if fire_ahead:
                fire((g + _NGB - 1) % _NGB, j + _NGB - 1)
            if wait_prev:
                with jax.named_scope("scat_wait"):
                    pltpu.make_async_copy(
                        tbufs[b], out_hbm.at[0, :, 0], ssems[b]).wait()
            with jax.named_scope("gath_wait"):
                pltpu.make_async_copy(
                    table_hbm.at[pl.ds(0, _BW)], gbufs[g], gsems[g]).wait()
            with jax.named_scope("transp"):
                transpose(g, b)
            pltpu.async_copy(tbufs[b], out_hbm.at[h, :, bwg], ssems[b])

        for g in range(_NGB - 1):
            fire(g, g)
        for j in range(_NBUF):
            step(j, j % _NGB, j % _NBUF, True, False)
        for j in range(_NBUF, _NGB):
            step(j, j % _NGB, j % _NBUF, True, True)

        def body(t, carry):
            j0 = _NGB * t
            for u in range(_NGB):
                step(j0 + u, u, u % _NBUF, True, True)
            return carry

        lax.fori_loop(1, nblk // _NGB - 1, body, 0)
        for j in range(nblk - _NGB, nblk):
            step(j, j % _NGB, j % _NBUF, j + _NGB - 1 < nblk, True)
        for b in range(_NBUF):
            pltpu.make_async_copy(
                tbufs[b], out_hbm.at[0, :, 0], ssems[b]).wait()

    return gather_kernel


def kernel(x, table):
    batch, hist = x.shape
    d_embed = table.shape[1]
    nwin = batch // _BW
    xt = x.astype(jnp.int32).reshape(nwin, _BW, hist).transpose(0, 2, 1)
    q = _make_kernel(batch, hist, d_embed)(xt, table)
    return q.transpose(2, 4, 0, 1, 3).reshape(batch, hist, d_embed)


# speedup vs baseline: 1.1025x; 1.1025x over previous
import functools

import jax
import jax.numpy as jnp
from jax import lax
from jax.experimental import pallas as pl
from jax.experimental.pallas import tpu as pltpu
from jax.experimental.pallas import tpu_sc as plsc

_NC = 2
_NS = 16
_NW = _NC * _NS
_BW = 128
_NBUF = 2
_NGB = 4


@functools.lru_cache(maxsize=None)
def _make_kernel(batch, hist, d_embed):
    nwin = batch // _BW
    wpt = nwin // _NW
    nblk = wpt * hist
    td_n = d_embed // 8
    mesh = plsc.VectorSubcoreMesh(core_axis_name="c", subcore_axis_name="s")

    @functools.partial(
        pl.kernel,
        mesh=mesh,
        out_type=jax.ShapeDtypeStruct((hist, td_n, nwin, 8, _BW), jnp.float32),
        scratch_types=[
            pltpu.VMEM((wpt, hist, _BW), jnp.int32),
            *[pltpu.VMEM((_BW, d_embed), jnp.float32) for _ in range(_NGB)],
            *[pltpu.VMEM((td_n, 8, _BW), jnp.float32) for _ in range(_NBUF)],
            *[pltpu.SemaphoreType.DMA for _ in range(_NGB + _NBUF)],
        ],
        compiler_params=pltpu.CompilerParams(
            use_tc_tiling_on_sc=False, needs_layout_passes=False),
    )
    def gather_kernel(idx_hbm, table_hbm, out_hbm, idx_v, *bufs_and_sems):
        gbufs = bufs_and_sems[:_NGB]
        tbufs = bufs_and_sems[_NGB:_NGB + _NBUF]
        gsems = bufs_and_sems[_NGB + _NBUF:2 * _NGB + _NBUF]
        ssems = bufs_and_sems[2 * _NGB + _NBUF:]
        wid = lax.axis_index("s") * _NC + lax.axis_index("c")
        pltpu.sync_copy(idx_hbm.at[pl.ds(wid * wpt, wpt)], idx_v)
        lanes = lax.iota(jnp.int32, 16)

        def fire(g, j):
            bw = j // hist
            h = j - bw * hist
            pltpu.async_copy(table_hbm.at[idx_v.at[bw, h]], gbufs[g], gsems[g])

        diag = [(lanes + k) % 16 for k in range(16)]
        diag_hi = [d // 8 for d in diag]
        diag_lo = [d % 8 for d in diag]

        def transpose(g, b):
            gbuf, tbuf = gbufs[g], tbufs[b]

            def kb_body(kb, carry):
                rows = lanes + kb * 16
                for d0 in range(0, d_embed, 16):
                    for k in range(16):
                        cols = diag[k] + d0
                        v = plsc.load_gather(gbuf, [rows, cols])
                        plsc.store_scatter(
                            tbuf, [diag_hi[k] + (d0 // 8), diag_lo[k], rows], v)
                return carry

            lax.fori_loop(0, _BW // 16, kb_body, 0)

        def step(j, g, b, fire_ahead, wait_prev):
            bw = j // hist
            h = j - bw * hist
            bwg = wid * wpt + bw
            if fire_ahead:
                fire((g + _NGB - 1) % _NGB, j + _NGB - 1)
            if wait_prev:
                with jax.named_scope("scat_wait"):
                    pltpu.make_async_copy(
                        tbufs[b], out_hbm.at[0, :, 0], ssems[b]).wait()
            with jax.named_scope("gath_wait"):
                pltpu.make_async_copy(
                    table_hbm.at[pl.ds(0, _BW)], gbufs[g], gsems[g]).wait()
            with jax.named_scope("transp"):
                transpose(g, b)
            pltpu.async_copy(tbufs[b], out_hbm.at[h, :, bwg], ssems[b])

        for g in range(_NGB - 1):
            fire(g, g)
        for j in range(_NBUF):
            step(j, j % _NGB, j % _NBUF, True, False)
        for j in range(_NBUF, _NGB):
            step(j, j % _NGB, j % _NBUF, True, True)

        def body(t, carry):
            j0 = _NGB * t
            for u in range(_NGB):
                step(j0 + u, u, u % _NBUF, True, True)
            return carry

        lax.fori_loop(1, nblk // _NGB - 1, body, 0)
        for j in range(nblk - _NGB, nblk):
            step(j, j % _NGB, j % _NBUF, j + _NGB - 1 < nblk, True)
        for b in range(_NBUF):
            pltpu.make_async_copy(
                tbufs[b], out_hbm.at[0, :, 0], ssems[b]).wait()

    return gather_kernel


def kernel(x, table):
    batch, hist = x.shape
    d_embed = table.shape[1]
    nwin = batch // _BW
    xt = x.astype(jnp.int32).reshape(nwin, _BW, hist).transpose(0, 2, 1)
    q = _make_kernel(batch, hist, d_embed)(xt, table)
    return q.transpose(2, 4, 0, 1, 3).reshape(batch, hist, d_embed)


# speedup vs baseline: 1.1535x; 1.0463x over previous
import functools

import jax
import jax.numpy as jnp
from jax import lax
from jax.experimental import pallas as pl
from jax.experimental.pallas import tpu as pltpu
from jax.experimental.pallas import tpu_sc as plsc

_NC = 2
_NS = 16
_NW = _NC * _NS
_BW = 128
_NBUF = 2
_NGB = 4


@functools.lru_cache(maxsize=None)
def _make_kernel(batch, hist, d_embed):
    nwin = batch // _BW
    wpt = nwin // _NW
    nblk = wpt * hist
    td_n = d_embed // 8
    mesh = plsc.VectorSubcoreMesh(core_axis_name="c", subcore_axis_name="s")

    @functools.partial(
        pl.kernel,
        mesh=mesh,
        out_type=jax.ShapeDtypeStruct((hist, td_n, nwin, 8, _BW), jnp.float32),
        scratch_types=[
            pltpu.VMEM((wpt, hist, _BW), jnp.int32),
            *[pltpu.VMEM((_BW, d_embed), jnp.float32) for _ in range(_NGB)],
            *[pltpu.VMEM((td_n, 8, _BW), jnp.float32) for _ in range(_NBUF)],
            *[pltpu.SemaphoreType.DMA for _ in range(_NGB + _NBUF)],
        ],
        compiler_params=pltpu.CompilerParams(
            use_tc_tiling_on_sc=False, needs_layout_passes=False),
    )
    def gather_kernel(idx_hbm, table_hbm, out_hbm, idx_v, *bufs_and_sems):
        gbufs = bufs_and_sems[:_NGB]
        tbufs = bufs_and_sems[_NGB:_NGB + _NBUF]
        gsems = bufs_and_sems[_NGB + _NBUF:2 * _NGB + _NBUF]
        ssems = bufs_and_sems[2 * _NGB + _NBUF:]
        wid = lax.axis_index("s") * _NC + lax.axis_index("c")
        pltpu.sync_copy(idx_hbm.at[pl.ds(wid * wpt, wpt)], idx_v)
        lanes = lax.iota(jnp.int32, 16)

        def fire(g, j):
            bw = j // hist
            h = j - bw * hist
            pltpu.async_copy(table_hbm.at[idx_v.at[bw, h]], gbufs[g], gsems[g])

        qv = lanes % 8
        pv = lanes // 8
        qv8 = qv * 8
        oct_base = 2 * qv + pv

        def transpose(g, b):
            gbuf, tbuf = gbufs[g], tbufs[b]

            def u_body(u, carry):
                octs = ((oct_base + u) % 16) * 8 + pv
                for s in range(4):
                    rows = octs + (2 * s)
                    for dsub in range(8):
                        cols = qv8 + dsub
                        v = plsc.load_gather(gbuf, [rows, cols])
                        plsc.store_scatter(
                            tbuf, [qv, jnp.full((16,), dsub, jnp.int32), rows],
                            v)
                return carry

            lax.fori_loop(0, 16, u_body, 0)

        def step(j, g, b, fire_ahead, wait_prev):
            bw = j // hist
            h = j - bw * hist
            bwg = wid * wpt + bw
            if fire_ahead:
                fire((g + _NGB - 1) % _NGB, j + _NGB - 1)
            if wait_prev:
                with jax.named_scope("scat_wait"):
                    pltpu.make_async_copy(
                        tbufs[b], out_hbm.at[0, :, 0], ssems[b]).wait()
            with jax.named_scope("gath_wait"):
                pltpu.make_async_copy(
                    table_hbm.at[pl.ds(0, _BW)], gbufs[g], gsems[g]).wait()
            with jax.named_scope("transp"):
                transpose(g, b)
            pltpu.async_copy(tbufs[b], out_hbm.at[h, :, bwg], ssems[b])

        for g in range(_NGB - 1):
            fire(g, g)
        for j in range(_NBUF):
            step(j, j % _NGB, j % _NBUF, True, False)
        for j in range(_NBUF, _NGB):
            step(j, j % _NGB, j % _NBUF, True, True)

        def body(t, carry):
            j0 = _NGB * t
            for u in range(_NGB):
                step(j0 + u, u, u % _NBUF, True, True)
            return carry

        lax.fori_loop(1, nblk // _NGB - 1, body, 0)
        for j in range(nblk - _NGB, nblk):
            step(j, j % _NGB, j % _NBUF, j + _NGB - 1 < nblk, True)
        for b in range(_NBUF):
            pltpu.make_async_copy(
                tbufs[b], out_hbm.at[0, :, 0], ssems[b]).wait()

    return gather_kernel


def kernel(x, table):
    batch, hist = x.shape
    d_embed = table.shape[1]
    nwin = batch // _BW
    xt = x.astype(jnp.int32).reshape(nwin, _BW, hist).transpose(0, 2, 1)
    q = _make_kernel(batch, hist, d_embed)(xt, table)
    return q.transpose(2, 4, 0, 1, 3).reshape(batch, hist, d_embed)
